# baseline (speedup 1.0000x reference)
import functools

import jax
import jax.numpy as jnp
from jax import lax
from jax.experimental import pallas as pl
from jax.experimental.pallas import tpu as pltpu
from jax.experimental.pallas import tpu_sc as plsc

NUM_T = 7
LANES = 16
NC, NS = 2, 16
NW = NC * NS
CHUNK = 8192
NBUF = 4


def _searchsorted_vreg(s, T):
    b2 = T[3] < s
    t1 = jnp.where(b2, T[5], T[1])
    b1 = t1 < s
    t0 = jnp.where(b2, jnp.where(b1, T[6], T[4]),
                   jnp.where(b1, T[2], T[0]))
    b0 = t0 < s
    four = jnp.full((LANES,), 4, jnp.int32)
    two = jnp.full((LANES,), 2, jnp.int32)
    one = jnp.full((LANES,), 1, jnp.int32)
    zero = jnp.zeros((LANES,), jnp.int32)
    return (jnp.where(b2, four, zero) + jnp.where(b1, two, zero)
            + jnp.where(b0, one, zero))


def _make_sc_kernel(n):
    per_w = n // NW
    chunks = per_w // CHUNK
    mesh = plsc.VectorSubcoreMesh(core_axis_name="c", subcore_axis_name="s")

    @functools.partial(
        pl.kernel,
        out_type=jax.ShapeDtypeStruct((n,), jnp.int32),
        mesh=mesh,
        scratch_types=[
            pltpu.VMEM((NUM_T, LANES), jnp.float32),
            pltpu.VMEM((NBUF, CHUNK), jnp.float32),
            pltpu.VMEM((NBUF, CHUNK), jnp.int32),
            pltpu.SemaphoreType.DMA((NBUF,)),
            pltpu.SemaphoreType.DMA((NBUF,)),
        ],
    )
    def body(scores_hbm, thr_hbm, out_hbm, thr_v, in_v, out_v, in_sem,
             out_sem):
        wid = lax.axis_index("s") * NC + lax.axis_index("c")
        base = wid * per_w
        pltpu.sync_copy(thr_hbm, thr_v)
        T = [thr_v[i] for i in range(NUM_T)]
        degenerate = T[0][0] == T[NUM_T - 1][0]

        def in_cp(ci, b):
            return pltpu.make_async_copy(
                scores_hbm.at[pl.ds(base + ci * CHUNK, CHUNK)],
                in_v.at[b], in_sem.at[b])

        def out_cp(ci, b):
            return pltpu.make_async_copy(
                out_v.at[b],
                out_hbm.at[pl.ds(base + ci * CHUNK, CHUNK)], out_sem.at[b])

        for b in range(NBUF):
            in_cp(b, b).start()

        def step(ci, b):
            in_cp(ci, b).wait()

            @pl.when(ci >= NBUF)
            def _():
                out_cp(ci - NBUF, b).wait()

            def fast_path():
                seven = jnp.full((LANES,), NUM_T, jnp.int32)
                zero = jnp.zeros((LANES,), jnp.int32)

                @plsc.parallel_loop(0, CHUNK, step=LANES, unroll=8)
                def _(vi):
                    s = in_v[b, pl.ds(vi, LANES)]
                    out_v[b, pl.ds(vi, LANES)] = jnp.where(T[0] < s, seven,
                                                           zero)

            def general_path():
                @plsc.parallel_loop(0, CHUNK, step=LANES, unroll=8)
                def _(vi):
                    s = in_v[b, pl.ds(vi, LANES)]
                    out_v[b, pl.ds(vi, LANES)] = _searchsorted_vreg(s, T)

            lax.cond(degenerate, fast_path, general_path)
            out_cp(ci, b).start()

            @pl.when(ci + NBUF < chunks)
            def _():
                in_cp(ci + NBUF, b).start()

        def outer(g, c):
            for b in range(NBUF):
                step(g * NBUF + b, b)
            return c

        lax.fori_loop(0, chunks // NBUF, outer, 0)
        for b in range(NBUF):
            out_cp(chunks - NBUF + b, b).wait()

    return body


def kernel(scores, thresholds):
    n = scores.shape[0]
    thr16 = jnp.broadcast_to(thresholds[:, None], (NUM_T, LANES))
    return _make_sc_kernel(n)(scores, thr16)


# speedup vs baseline: 1.0115x; 1.0115x over previous
import functools

import jax
import jax.numpy as jnp
from jax import lax
from jax.experimental import pallas as pl
from jax.experimental.pallas import tpu as pltpu
from jax.experimental.pallas import tpu_sc as plsc

NUM_T = 7
LANES = 16
NC, NS = 2, 16
NW = NC * NS
CHUNK = 16384
NBUF = 2


def _searchsorted_vreg(s, T):
    b2 = T[3] < s
    t1 = jnp.where(b2, T[5], T[1])
    b1 = t1 < s
    t0 = jnp.where(b2, jnp.where(b1, T[6], T[4]),
                   jnp.where(b1, T[2], T[0]))
    b0 = t0 < s
    four = jnp.full((LANES,), 4, jnp.int32)
    two = jnp.full((LANES,), 2, jnp.int32)
    one = jnp.full((LANES,), 1, jnp.int32)
    zero = jnp.zeros((LANES,), jnp.int32)
    return (jnp.where(b2, four, zero) + jnp.where(b1, two, zero)
            + jnp.where(b0, one, zero))


def _make_sc_kernel(n):
    per_w = n // NW
    chunks = per_w // CHUNK
    mesh = plsc.VectorSubcoreMesh(core_axis_name="c", subcore_axis_name="s")

    @functools.partial(
        pl.kernel,
        out_type=jax.ShapeDtypeStruct((n,), jnp.int32),
        mesh=mesh,
        scratch_types=[
            pltpu.VMEM((NUM_T, LANES), jnp.float32),
            pltpu.VMEM((NBUF, CHUNK), jnp.float32),
            pltpu.VMEM((NBUF, CHUNK), jnp.int32),
            pltpu.SemaphoreType.DMA((NBUF,)),
            pltpu.SemaphoreType.DMA((NBUF,)),
        ],
    )
    def body(scores_hbm, thr_hbm, out_hbm, thr_v, in_v, out_v, in_sem,
             out_sem):
        wid = lax.axis_index("s") * NC + lax.axis_index("c")
        base = wid * per_w
        pltpu.sync_copy(thr_hbm, thr_v)
        T = [thr_v[i] for i in range(NUM_T)]
        degenerate = T[0][0] == T[NUM_T - 1][0]

        def in_cp(ci, b):
            return pltpu.make_async_copy(
                scores_hbm.at[pl.ds(base + ci * CHUNK, CHUNK)],
                in_v.at[b], in_sem.at[b])

        def out_cp(ci, b):
            return pltpu.make_async_copy(
                out_v.at[b],
                out_hbm.at[pl.ds(base + ci * CHUNK, CHUNK)], out_sem.at[b])

        for b in range(NBUF):
            in_cp(b, b).start()

        def step(ci, b):
            in_cp(ci, b).wait()

            @pl.when(ci >= NBUF)
            def _():
                out_cp(ci - NBUF, b).wait()

            def fast_path():
                seven = jnp.full((LANES,), NUM_T, jnp.int32)
                zero = jnp.zeros((LANES,), jnp.int32)

                @plsc.parallel_loop(0, CHUNK, step=LANES, unroll=8)
                def _(vi):
                    s = in_v[b, pl.ds(vi, LANES)]
                    out_v[b, pl.ds(vi, LANES)] = jnp.where(T[0] < s, seven,
                                                           zero)

            def general_path():
                @plsc.parallel_loop(0, CHUNK, step=LANES, unroll=8)
                def _(vi):
                    s = in_v[b, pl.ds(vi, LANES)]
                    out_v[b, pl.ds(vi, LANES)] = _searchsorted_vreg(s, T)

            lax.cond(degenerate, fast_path, general_path)
            out_cp(ci, b).start()

            @pl.when(ci + NBUF < chunks)
            def _():
                in_cp(ci + NBUF, b).start()

        def outer(g, c):
            for b in range(NBUF):
                step(g * NBUF + b, b)
            return c

        lax.fori_loop(0, chunks // NBUF, outer, 0)
        for b in range(NBUF):
            out_cp(chunks - NBUF + b, b).wait()

    return body


def kernel(scores, thresholds):
    n = scores.shape[0]
    thr16 = jnp.broadcast_to(thresholds[:, None], (NUM_T, LANES))
    return _make_sc_kernel(n)(scores, thr16)
